# baseline (speedup 1.0000x reference)
import jax
import jax.numpy as jnp
from jax.experimental import pallas as pl
from jax.experimental.pallas import tpu as pltpu

_MAX_REL = 128
_EMB = 64
_LEN = 2048
_TAB = 2 * _MAX_REL + 1
_EXT_PAD = 2 * _LEN
_ROWS_PER_STEP = 16
_STEPS = _LEN // _ROWS_PER_STEP


def _build_frev_kernel(w_ref, frev_hbm, frev_vmem, sem):
    top = _LEN - _MAX_REL - 1
    frev_vmem[0:top, :] = jnp.broadcast_to(
        w_ref[_TAB - 1:_TAB, :], (top, _EMB))
    frev_vmem[top + _TAB:_EXT_PAD, :] = jnp.broadcast_to(
        w_ref[0:1, :], (_EXT_PAD - top - _TAB, _EMB))
    for j in range(_TAB):
        frev_vmem[top + j:top + j + 1, :] = w_ref[_TAB - 1 - j:_TAB - j, :]
    pltpu.make_async_copy(frev_vmem, frev_hbm, sem).start()
    pltpu.make_async_copy(frev_vmem, frev_hbm, sem).wait()


def _stream_kernel(frev_hbm, out_ref, sems):
    k = pl.program_id(0)

    def copy_for(row, bank, r):
        return pltpu.make_async_copy(
            frev_hbm.at[pl.ds(_LEN - 1 - row, _LEN), :],
            out_ref.at[row],
            sems.at[bank, r],
        )

    bank = jax.lax.rem(k, 2)
    for r in range(_ROWS_PER_STEP):
        copy_for(k * _ROWS_PER_STEP + r, bank, r).start()

    @pl.when(k > 0)
    def _wait_prev():
        for r in range(_ROWS_PER_STEP):
            copy_for((k - 1) * _ROWS_PER_STEP + r, 1 - bank, r).wait()

    @pl.when(k == _STEPS - 1)
    def _wait_last():
        for r in range(_ROWS_PER_STEP):
            copy_for(k * _ROWS_PER_STEP + r, bank, r).wait()


@jax.jit
def _run(W):
    frev = pl.pallas_call(
        _build_frev_kernel,
        in_specs=[pl.BlockSpec((_TAB, _EMB), lambda: (0, 0))],
        out_specs=pl.BlockSpec(memory_space=pl.ANY),
        out_shape=jax.ShapeDtypeStruct((_EXT_PAD, _EMB), jnp.float32),
        scratch_shapes=[
            pltpu.VMEM((_EXT_PAD, _EMB), jnp.float32),
            pltpu.SemaphoreType.DMA,
        ],
    )(W)
    return pl.pallas_call(
        _stream_kernel,
        grid=(_STEPS,),
        in_specs=[pl.BlockSpec(memory_space=pl.ANY)],
        out_specs=pl.BlockSpec(memory_space=pl.ANY),
        out_shape=jax.ShapeDtypeStruct((_LEN, _LEN, _EMB), jnp.float32),
        scratch_shapes=[
            pltpu.SemaphoreType.DMA((2, _ROWS_PER_STEP)),
        ],
    )(frev)


def kernel(W, length):
    return _run(W)


# speedup vs baseline: 29.7107x; 29.7107x over previous
import jax
import jax.numpy as jnp
from jax import lax
from jax.experimental import pallas as pl
from jax.experimental.pallas import tpu as pltpu
from jax.experimental.pallas import tpu_sc as plsc

_MAX_REL = 128
_EMB = 64
_LEN = 2048
_TAB = 2 * _MAX_REL + 1
_EXT_PAD = 2 * _LEN + 8
_FLAT = _EXT_PAD * _EMB
_W2 = 2 * _EMB
_ROW2D = _LEN * _EMB // _W2
_NC = 2
_NS = 16
_ROWS_PER_WORKER = _LEN // (_NC * _NS)


def _build_frev_kernel(w_ref, frev_ref):
    top = _LEN - _MAX_REL - 1
    frev_ref[0:top, :] = jnp.broadcast_to(
        w_ref[_TAB - 1:_TAB, :], (top, _EMB))
    frev_ref[top + _TAB:_EXT_PAD, :] = jnp.broadcast_to(
        w_ref[0:1, :], (_EXT_PAD - top - _TAB, _EMB))
    for j in range(_TAB):
        frev_ref[top + j:top + j + 1, :] = w_ref[_TAB - 1 - j:_TAB - j, :]


def _sc_stream_body(a_hbm, b_hbm, out_hbm, a_sh, b_sh, sem):
    c = lax.axis_index("c")
    s = lax.axis_index("s")

    @pl.when(s == 0)
    def _stage():
        pltpu.sync_copy(a_hbm, a_sh)
        pltpu.sync_copy(b_hbm, b_sh)

    plsc.subcore_barrier()

    wid = s * _NC + c
    base_row = wid * _ROWS_PER_WORKER
    descs = []
    for t in range(_ROWS_PER_WORKER):
        row = base_row + t
        q = (_LEN - 1 - row) // 2
        src = b_sh if t % 2 == 0 else a_sh
        descs.append(pltpu.async_copy(
            src.at[pl.ds(q, _ROW2D), :],
            out_hbm.at[row],
            sem,
        ))
    for d in descs:
        d.wait()


@jax.jit
def _run(W):
    frev = pl.pallas_call(
        _build_frev_kernel,
        in_specs=[pl.BlockSpec((_TAB, _EMB), lambda: (0, 0))],
        out_specs=pl.BlockSpec((_EXT_PAD, _EMB), lambda: (0, 0)),
        out_shape=jax.ShapeDtypeStruct((_EXT_PAD, _EMB), jnp.float32),
    )(W)
    flat = frev.reshape(_FLAT)
    a2d = flat[:_LEN * _W2].reshape(_LEN, _W2)
    b2d = flat[_EMB:_EMB + _LEN * _W2].reshape(_LEN, _W2)

    sc_call = pl.kernel(
        _sc_stream_body,
        out_type=jax.ShapeDtypeStruct((_LEN, _ROW2D, _W2), jnp.float32),
        mesh=plsc.VectorSubcoreMesh(
            core_axis_name="c", subcore_axis_name="s"),
        scratch_types=[
            pltpu.MemorySpace.VMEM_SHARED((_LEN, _W2), jnp.float32),
            pltpu.MemorySpace.VMEM_SHARED((_LEN, _W2), jnp.float32),
            pltpu.SemaphoreType.DMA,
        ],
    )
    out = sc_call(a2d, b2d)
    return out.reshape(_LEN, _LEN, _EMB)


def kernel(W, length):
    return _run(W)
